# speedup vs baseline: 1.5740x; 1.5740x over previous
import jax
import jax.numpy as jnp
from jax.experimental import pallas as pl
from jax.experimental.pallas import tpu as pltpu

_K = 8192
_D = 32
_N = 8192
_BT = 256
_NBLK = _N // _BT
_COMMIT = 0.25
_ENT_RATIO = 0.1
_TEMP = 0.01


def _vq_block(x_ref, cb_ref, quant_ref, idx_ref, loss_ref, acc_ref, sums_ref):
    i = pl.program_id(0)

    @pl.when(i == 0)
    def _init():
        acc_ref[...] = jnp.zeros_like(acc_ref)
        sums_ref[0] = 0.0
        sums_ref[1] = 0.0

    xb = x_ref[...]
    cb = cb_ref[...]
    a2 = jnp.sum(xb * xb, axis=1, keepdims=True)
    b2 = jnp.sum(cb * cb, axis=1)[None, :]
    ab = jax.lax.dot_general(xb, cb, (((1,), (1,)), ((), ())),
                             preferred_element_type=jnp.float32)
    d = a2 - 2 * ab + b2

    iota = jax.lax.broadcasted_iota(jnp.int32, d.shape, 1)
    dmin = jnp.min(d, axis=1, keepdims=True)
    idx = jnp.min(jnp.where(d == dmin, iota, _K), axis=1).astype(jnp.int32)
    idx_ref[0, 0, :] = idx

    onehot = (iota == idx[:, None]).astype(jnp.float32)
    quant = jax.lax.dot_general(onehot, cb, (((1,), (0,)), ((), ())),
                                preferred_element_type=jnp.float32)
    quant_ref[...] = quant

    l = (-d) / _TEMP
    m = jnp.max(l, axis=1, keepdims=True)
    e = jnp.exp(l - m)
    s = jnp.sum(e, axis=1, keepdims=True)
    t = jnp.sum(e * (l - m), axis=1, keepdims=True)
    plogp = t / s - jnp.log(s)
    acc_ref[...] += jnp.sum(e / s, axis=0, keepdims=True)
    sums_ref[0] += jnp.sum(plogp)
    sums_ref[1] += jnp.sum((quant - xb) ** 2)

    @pl.when(i == _NBLK - 1)
    def _fin():
        avg = acc_ref[...] / _N
        avg_ent = -jnp.sum(avg * jnp.log(avg + 1e-5))
        sample_ent = -(sums_ref[0] / _N)
        mse = sums_ref[1] / (_N * _D)
        loss_ref[0, 0] = (mse * _COMMIT + mse
                          + _ENT_RATIO * (sample_ent - avg_ent))


def kernel(x, codebook):
    codebook = jnp.asarray(codebook, dtype=jnp.float32)
    xf = jnp.reshape(x, (-1, _D))
    quant, idx, loss = pl.pallas_call(
        _vq_block,
        grid=(_NBLK,),
        in_specs=[
            pl.BlockSpec((_BT, _D), lambda i: (i, 0)),
            pl.BlockSpec((_K, _D), lambda i: (0, 0)),
        ],
        out_specs=[
            pl.BlockSpec((_BT, _D), lambda i: (i, 0)),
            pl.BlockSpec((1, 1, _BT), lambda i: (i, 0, 0)),
            pl.BlockSpec(memory_space=pltpu.SMEM, block_shape=(1, 1),
                         index_map=lambda i: (0, 0)),
        ],
        out_shape=[
            jax.ShapeDtypeStruct((_N, _D), jnp.float32),
            jax.ShapeDtypeStruct((_NBLK, 1, _BT), jnp.int32),
            jax.ShapeDtypeStruct((1, 1), jnp.float32),
        ],
        scratch_shapes=[
            pltpu.VMEM((1, _K), jnp.float32),
            pltpu.SMEM((2,), jnp.float32),
        ],
    )(xf, codebook)
    quantized = jnp.reshape(quant, x.shape)
    encoding_indices = jnp.reshape(idx, x.shape[:-1])
    return quantized, loss[0, 0], encoding_indices
